# baseline (speedup 1.0000x reference)
import functools

import jax
import jax.numpy as jnp
from jax import lax
from jax.experimental import pallas as pl
from jax.experimental.pallas import tpu as pltpu
from jax.experimental.pallas import tpu_sc as plsc

DIM = 64
LANES = 16
IDX_CHUNK = 128


@functools.lru_cache(maxsize=None)
def _make_sc_kernel(B, n_rows):
    info = plsc.get_sparse_core_info()
    NC, NS = info.num_cores, info.num_subcores
    NW = NC * NS
    bw = B // NW
    n_idx = bw // IDX_CHUNK
    n_sub = bw // LANES
    rpd = n_rows // LANES
    glist = DIM * LANES
    mesh = plsc.VectorSubcoreMesh(core_axis_name="c", subcore_axis_name="s")

    @functools.partial(
        pl.kernel,
        mesh=mesh,
        compiler_params=pltpu.CompilerParams(
            needs_layout_passes=False, use_tc_tiling_on_sc=False),
        out_type=(
            jax.ShapeDtypeStruct((B,), jnp.float32),
            jax.ShapeDtypeStruct((B,), jnp.float32),
        ),
        scratch_types=[
            pltpu.VMEM((n_idx, IDX_CHUNK), jnp.int32),
            pltpu.VMEM((n_idx, IDX_CHUNK), jnp.int32),
            pltpu.VMEM((n_idx, IDX_CHUNK), jnp.int32),
            pltpu.VMEM((DIM // 8, IDX_CHUNK), jnp.int32),
            pltpu.VMEM((DIM // 8, IDX_CHUNK), jnp.int32),
            pltpu.VMEM((DIM // 8, IDX_CHUNK), jnp.int32),
            pltpu.VMEM((glist, LANES), jnp.float32),
            pltpu.VMEM((glist, LANES), jnp.float32),
            pltpu.VMEM((glist, LANES), jnp.float32),
            pltpu.VMEM((bw,), jnp.float32),
            pltpu.VMEM((bw,), jnp.float32),
            pltpu.SemaphoreType.DMA,
        ],
    )
    def sc_kernel(u_hbm, i_hbm, n_hbm, ut_hbm, it_hbm, pos_hbm, neg_hbm,
                  uidx, iidx, nidx, ulist, ilist, nlist,
                  ubuf, ibuf, nbuf, opos, oneg, sem):
        wid = lax.axis_index("s") * NC + lax.axis_index("c")
        base = wid * bw

        idx_cps = []
        for j in range(n_idx):
            src = pl.ds(base + j * IDX_CHUNK, IDX_CHUNK)
            idx_cps.append(pltpu.async_copy(u_hbm.at[src], uidx.at[j], sem))
            idx_cps.append(pltpu.async_copy(i_hbm.at[src], iidx.at[j], sem))
            idx_cps.append(pltpu.async_copy(n_hbm.at[src], nidx.at[j], sem))
        for c in idx_cps:
            c.wait()

        iota = lax.iota(jnp.int32, LANES)
        tables = ((uidx, ulist, ubuf, ut_hbm),
                  (iidx, ilist, ibuf, it_hbm),
                  (nidx, nlist, nbuf, it_hbm))

        def body(s, carry):
            j = s // (IDX_CHUNK // LANES)
            o = (s % (IDX_CHUNK // LANES)) * LANES

            cps = []
            offs = []
            for idx_ref, list_ref, buf_ref, t_hbm in tables:
                idxv = idx_ref[j, pl.ds(o, LANES)]
                r4 = lax.shift_right_logical(idxv, 4)
                offs.append(lax.bitwise_and(idxv, 15))
                for d in range(DIM):
                    list_ref[d // 8, pl.ds((d % 8) * LANES, LANES)] = (
                        r4 + d * rpd)
                for r in range(DIM // 8):
                    cps.append(pltpu.async_copy(
                        t_hbm.at[list_ref.at[r]],
                        buf_ref.at[pl.ds(r * IDX_CHUNK, IDX_CHUNK)], sem))
            for c in cps:
                c.wait()

            uoff, ioff, noff = offs
            ap0 = jnp.zeros((LANES,), jnp.float32)
            ap1 = jnp.zeros((LANES,), jnp.float32)
            an0 = jnp.zeros((LANES,), jnp.float32)
            an1 = jnp.zeros((LANES,), jnp.float32)
            for d in range(DIM):
                row = iota + d * LANES
                uv = plsc.load_gather(ubuf, [row, uoff])
                iv = plsc.load_gather(ibuf, [row, ioff])
                nv = plsc.load_gather(nbuf, [row, noff])
                if d % 2 == 0:
                    ap0 = ap0 + uv * iv
                    an0 = an0 + uv * nv
                else:
                    ap1 = ap1 + uv * iv
                    an1 = an1 + uv * nv
            out = pl.ds(s * LANES, LANES)
            opos[out] = ap0 + ap1
            oneg[out] = an0 + an1
            return carry

        lax.fori_loop(0, n_sub, body, 0)

        pltpu.sync_copy(opos, pos_hbm.at[pl.ds(base, bw)])
        pltpu.sync_copy(oneg, neg_hbm.at[pl.ds(base, bw)])

    return sc_kernel


def kernel(u, i, neg_i, user_emb, item_emb):
    B = u.shape[0]
    n_rows = user_emb.shape[0]
    sc = _make_sc_kernel(B, n_rows)
    ut = user_emb.T.reshape(-1, LANES)
    it = item_emb.T.reshape(-1, LANES)
    return sc(u.astype(jnp.int32), i.astype(jnp.int32),
              neg_i.astype(jnp.int32), ut, it)


# speedup vs baseline: 10.6098x; 10.6098x over previous
import functools

import jax
import jax.numpy as jnp
from jax import lax
from jax.experimental import pallas as pl
from jax.experimental.pallas import tpu as pltpu
from jax.experimental.pallas import tpu_sc as plsc

DIM = 64
LANES = 16
CHUNK = 128


@functools.lru_cache(maxsize=None)
def _make_sc_kernel(B):
    info = plsc.get_sparse_core_info()
    NC, NS = info.num_cores, info.num_subcores
    NW = NC * NS
    bw = B // NW
    n_chunk = bw // CHUNK
    n_grp = CHUNK // LANES
    mesh = plsc.VectorSubcoreMesh(core_axis_name="c", subcore_axis_name="s")

    @functools.partial(
        pl.kernel,
        mesh=mesh,
        compiler_params=pltpu.CompilerParams(needs_layout_passes=False),
        out_type=(
            jax.ShapeDtypeStruct((B,), jnp.float32),
            jax.ShapeDtypeStruct((B,), jnp.float32),
        ),
        scratch_types=[
            pltpu.VMEM((n_chunk, CHUNK), jnp.int32),
            pltpu.VMEM((n_chunk, CHUNK), jnp.int32),
            pltpu.VMEM((n_chunk, CHUNK), jnp.int32),
            pltpu.VMEM((2 * CHUNK, 2 * DIM), jnp.float32),
            pltpu.VMEM((2 * CHUNK, 2 * DIM), jnp.float32),
            pltpu.VMEM((2 * CHUNK, 2 * DIM), jnp.float32),
            pltpu.VMEM((bw,), jnp.float32),
            pltpu.VMEM((bw,), jnp.float32),
            pltpu.SemaphoreType.DMA,
            pltpu.SemaphoreType.DMA,
            pltpu.SemaphoreType.DMA,
        ],
    )
    def sc_kernel(u_hbm, i_hbm, n_hbm, cat_hbm, pos_hbm, neg_hbm,
                  uidx, iidx, nidx, ubuf, ibuf, nbuf, opos, oneg,
                  semi, sem0, sem1):
        wid = lax.axis_index("s") * NC + lax.axis_index("c")
        base = wid * bw

        idx_cps = []
        for j in range(n_chunk):
            src = pl.ds(base + j * CHUNK, CHUNK)
            idx_cps.append(pltpu.async_copy(u_hbm.at[src], uidx.at[j], semi))
            idx_cps.append(pltpu.async_copy(i_hbm.at[src], iidx.at[j], semi))
            idx_cps.append(pltpu.async_copy(n_hbm.at[src], nidx.at[j], semi))
        for c in idx_cps:
            c.wait()

        sems = (sem0, sem1)

        def fire(c):
            slot = c % 2
            dst = pl.ds(slot * CHUNK, CHUNK)
            sem = sems[slot]
            return [
                pltpu.async_copy(cat_hbm.at[uidx.at[c]], ubuf.at[dst], sem),
                pltpu.async_copy(cat_hbm.at[iidx.at[c]], ibuf.at[dst], sem),
                pltpu.async_copy(cat_hbm.at[nidx.at[c]], nbuf.at[dst], sem),
            ]

        iota = lax.iota(jnp.int32, LANES)
        inflight = fire(0)
        for c in range(n_chunk):
            for cp in inflight:
                cp.wait()
            if c + 1 < n_chunk:
                inflight = fire(c + 1)
            slot = c % 2

            def body(g, carry, c=c, slot=slot):
                brow = iota + (slot * CHUNK + g * LANES)
                ap0 = jnp.zeros((LANES,), jnp.float32)
                ap1 = jnp.zeros((LANES,), jnp.float32)
                an0 = jnp.zeros((LANES,), jnp.float32)
                an1 = jnp.zeros((LANES,), jnp.float32)
                for d in range(DIM):
                    ucol = jnp.full((LANES,), d, jnp.int32)
                    icol = jnp.full((LANES,), DIM + d, jnp.int32)
                    uv = plsc.load_gather(ubuf, [brow, ucol])
                    iv = plsc.load_gather(ibuf, [brow, icol])
                    nv = plsc.load_gather(nbuf, [brow, icol])
                    if d % 2 == 0:
                        ap0 = ap0 + uv * iv
                        an0 = an0 + uv * nv
                    else:
                        ap1 = ap1 + uv * iv
                        an1 = an1 + uv * nv
                out = pl.ds((c * n_grp + g) * LANES, LANES)
                opos[out] = ap0 + ap1
                oneg[out] = an0 + an1
                return carry

            lax.fori_loop(0, n_grp, body, 0)

        pltpu.sync_copy(opos, pos_hbm.at[pl.ds(base, bw)])
        pltpu.sync_copy(oneg, neg_hbm.at[pl.ds(base, bw)])

    return sc_kernel


def kernel(u, i, neg_i, user_emb, item_emb):
    B = u.shape[0]
    sc = _make_sc_kernel(B)
    cat = jnp.concatenate([user_emb, item_emb], axis=1)
    return sc(u.astype(jnp.int32), i.astype(jnp.int32),
              neg_i.astype(jnp.int32), cat)
